# Initial kernel scaffold; baseline (speedup 1.0000x reference)
import jax
import jax.numpy as jnp
from jax.experimental import pallas as pl


def kernel(feat0, feat1, feat2):
    raise NotImplementedError("write your pallas kernel here")


# speedup vs baseline: 1.7910x; 1.7910x over previous
import jax
import jax.numpy as jnp
import numpy as np
from jax import lax
from jax.experimental import pallas as pl
from jax.experimental.pallas import tpu as pltpu

_NCLS = 80
_SCALES = (52, 26, 13)
_ANCHORS = (
    np.array([[10., 13.], [16., 30.], [33., 23.]], dtype=np.float32),
    np.array([[30., 61.], [62., 45.], [59., 119.]], dtype=np.float32),
    np.array([[116., 90.], [156., 198.], [373., 326.]], dtype=np.float32),
)
_T = sum(3 * h * h for h in _SCALES)
_TP = 10752
_NCH = _TP // 128
_CONF = 0.3
_IOU = 0.5
_K = 100


def _build_consts():
    gx, gy, aw, ah, hh = [], [], [], [], []
    for s, h in enumerate(_SCALES):
        r = np.arange(h * h)
        gx.append(np.repeat((r % h).astype(np.float32), 3))
        gy.append(np.repeat((r // h).astype(np.float32), 3))
        a = _ANCHORS[s].astype(np.float32) / np.float32(416.0)
        aw.append(np.tile(a[:, 0], h * h))
        ah.append(np.tile(a[:, 1], h * h))
        hh.append(np.full(3 * h * h, h, np.float32))

    def cat(v, pad=0.0):
        out = np.full(_TP, pad, np.float32)
        out[:_T] = np.concatenate(v)
        return out

    valid = np.zeros(_TP, np.float32)
    valid[:_T] = 1.0
    zero = np.zeros(_TP, np.float32)
    return np.stack([cat(gx), cat(gy), cat(aw), cat(ah),
                     cat(hh, pad=1.0), valid, zero, zero])


_G = _build_consts()


def _body(cls_ref, txy_ref, g_ref, os_ref, ob_ref, oc_ref,
          s_ref, b_ref, ts_ref, x1_ref, y1_ref, x2_ref, y2_ref, ar_ref):
    f32 = jnp.float32
    lane = lax.broadcasted_iota(jnp.int32, (_NCLS, 128), 1)
    row80 = lax.broadcasted_iota(jnp.int32, (_NCLS, 128), 0)
    BIG = jnp.int32(2 ** 30)

    def decode_chunk(c, _):
        off = pl.multiple_of(c * 128, 128)
        cl = cls_ref[:, pl.ds(off, 128)]
        t = txy_ref[:, pl.ds(off, 128)]
        g = g_ref[:, pl.ds(off, 128)]
        conf = jax.nn.sigmoid(t[4:5, :])
        sc = conf * jax.nn.sigmoid(cl)
        s_ref[:, pl.ds(off, 128)] = jnp.where(g[5:6, :] > 0.0, sc, -2.0)
        bx = (g[0:1, :] + jax.nn.sigmoid(t[0:1, :])) / g[4:5, :]
        by = (g[1:2, :] + jax.nn.sigmoid(t[1:2, :])) / g[4:5, :]
        bw = jnp.exp(t[2:3, :]) * g[2:3, :]
        bh = jnp.exp(t[3:4, :]) * g[3:4, :]
        x1 = bx - bw / 2.0
        y1 = by - bh / 2.0
        x2 = bx + bw / 2.0
        y2 = by + bh / 2.0
        z = jnp.zeros_like(x1)
        b_ref[:, pl.ds(off, 128)] = jnp.concatenate(
            [x1, y1, x2, y2, z, z, z, z], axis=0)
        return 0

    lax.fori_loop(0, _NCH, decode_chunk, 0, unroll=2)

    ts_ref[...] = jnp.full((_NCLS, 128), -3.0, f32)
    zf = jnp.zeros((_NCLS, 128), f32)
    x1_ref[...] = zf
    y1_ref[...] = zf
    x2_ref[...] = zf
    y2_ref[...] = zf
    ar_ref[...] = zf

    def extract_body(i, carry_o):
        keepm, gprev = carry_o

        def scan_chunk(c, carry):
            vmax, vch = carry
            off = pl.multiple_of(c * 128, 128)
            ch = s_ref[:, pl.ds(off, 128)]
            ch = jnp.where(gprev - off == lane, jnp.float32(-2.0), ch)
            s_ref[:, pl.ds(off, 128)] = ch
            better = ch > vmax
            return (jnp.where(better, ch, vmax),
                    jnp.where(better, c, vch))

        vmax, vch = lax.fori_loop(
            0, _NCH, scan_chunk,
            (jnp.full((_NCLS, 128), -1e30, f32),
             jnp.zeros((_NCLS, 128), jnp.int32)), unroll=7)
        m = jnp.max(vmax, axis=1, keepdims=True)
        eq = vmax == m
        cstar = jnp.min(jnp.where(eq, vch, BIG), axis=1, keepdims=True)
        lstar = jnp.min(jnp.where(eq & (vch == cstar), lane, BIG),
                        axis=1, keepdims=True)
        gcur = jnp.broadcast_to(cstar * 128 + lstar, (_NCLS, 128))

        def gather_chunk(c, acc):
            ax1, ay1, ax2, ay2 = acc
            off = pl.multiple_of(c * 128, 128)
            bb = b_ref[:, pl.ds(off, 128)]
            loc = gcur - off == lane
            return (jnp.where(loc, bb[0:1, :], ax1),
                    jnp.where(loc, bb[1:2, :], ay1),
                    jnp.where(loc, bb[2:3, :], ax2),
                    jnp.where(loc, bb[3:4, :], ay2))

        ax1, ay1, ax2, ay2 = lax.fori_loop(
            0, _NCH, gather_chunk, (zf, zf, zf, zf), unroll=7)
        xi1 = jnp.sum(ax1, axis=1, keepdims=True)
        yi1 = jnp.sum(ay1, axis=1, keepdims=True)
        xi2 = jnp.sum(ax2, axis=1, keepdims=True)
        yi2 = jnp.sum(ay2, axis=1, keepdims=True)
        area_i = (xi2 - xi1) * (yi2 - yi1)

        xx1 = jnp.maximum(xi1, x1_ref[...])
        yy1 = jnp.maximum(yi1, y1_ref[...])
        xx2 = jnp.minimum(xi2, x2_ref[...])
        yy2 = jnp.minimum(yi2, y2_ref[...])
        inter = jnp.maximum(xx2 - xx1, 0.0) * jnp.maximum(yy2 - yy1, 0.0)
        iou = inter / (area_i + ar_ref[...] - inter + 1e-9)
        suppedf = jnp.max(
            jnp.where((keepm > 0.5) & (iou > _IOU), 1.0, 0.0),
            axis=1, keepdims=True)
        valid = (suppedf < 0.5) & (m > _CONF)

        colm = lane == i
        keepm = jnp.where(colm & valid, 1.0, keepm)
        ts_ref[...] = jnp.where(colm, m, ts_ref[...])
        x1_ref[...] = jnp.where(colm, xi1, x1_ref[...])
        y1_ref[...] = jnp.where(colm, yi1, y1_ref[...])
        x2_ref[...] = jnp.where(colm, xi2, x2_ref[...])
        y2_ref[...] = jnp.where(colm, yi2, y2_ref[...])
        ar_ref[...] = jnp.where(colm, area_i, ar_ref[...])
        return keepm, gcur

    keepm, _ = lax.fori_loop(
        0, _K, extract_body,
        (jnp.zeros((_NCLS, 128), jnp.float32),
         jnp.full((_NCLS, 128), -(2 ** 30), jnp.int32)))

    V = jnp.where(keepm > 0.5, ts_ref[...],
                  jnp.where(lane < _K, jnp.float32(-1.0), jnp.float32(-3.0)))
    row8 = lax.broadcasted_iota(jnp.int32, (8, 128), 0)
    col8 = lax.broadcasted_iota(jnp.int32, (8, 128), 1)

    def merge_body(k, carry):
        V, OS, OB, OC = carry
        mv = jnp.max(V)
        eqv = V == mv
        rstar = jnp.min(jnp.where(eqv, row80, BIG))
        ohr = eqv & (row80 == rstar)
        lst = jnp.min(jnp.where(ohr, lane, BIG))
        oh = ohr & (lane == lst)
        ohf = jnp.where(oh, jnp.float32(1.0), jnp.float32(0.0))
        bx1 = jnp.sum(ohf * x1_ref[...])
        by1 = jnp.sum(ohf * y1_ref[...])
        bx2 = jnp.sum(ohf * x2_ref[...])
        by2 = jnp.sum(ohf * y2_ref[...])
        selc = col8 == k
        OS = jnp.where(selc & (row8 == 0), mv, OS)
        OC = jnp.where(selc & (row8 == 0), rstar, OC)
        OB = jnp.where(selc & (row8 == 0), bx1, OB)
        OB = jnp.where(selc & (row8 == 1), by1, OB)
        OB = jnp.where(selc & (row8 == 2), bx2, OB)
        OB = jnp.where(selc & (row8 == 3), by2, OB)
        V = jnp.where(oh, jnp.float32(-5.0), V)
        return V, OS, OB, OC

    z8 = jnp.zeros((8, 128), f32)
    _, OS, OB, OC = lax.fori_loop(
        0, _K, merge_body, (V, z8, z8, jnp.zeros((8, 128), jnp.int32)))
    os_ref[...] = OS
    ob_ref[...] = OB
    oc_ref[...] = OC


def kernel(feat0, feat1, feat2):
    cols = [f.reshape(-1, _NCLS + 5).T for f in (feat0, feat1, feat2)]
    F = jnp.concatenate(cols, axis=1)
    F = jnp.pad(F, ((0, 0), (0, _TP - _T)))
    CLS = F[5:, :]
    TXY = jnp.pad(F[:5, :], ((0, 3), (0, 0)))
    G = jnp.asarray(_G)

    os_, ob, oc = pl.pallas_call(
        _body,
        out_shape=[jax.ShapeDtypeStruct((8, 128), jnp.float32),
                   jax.ShapeDtypeStruct((8, 128), jnp.float32),
                   jax.ShapeDtypeStruct((8, 128), jnp.int32)],
        scratch_shapes=[pltpu.VMEM((_NCLS, _TP), jnp.float32),
                        pltpu.VMEM((8, _TP), jnp.float32),
                        pltpu.VMEM((_NCLS, 128), jnp.float32),
                        pltpu.VMEM((_NCLS, 128), jnp.float32),
                        pltpu.VMEM((_NCLS, 128), jnp.float32),
                        pltpu.VMEM((_NCLS, 128), jnp.float32),
                        pltpu.VMEM((_NCLS, 128), jnp.float32),
                        pltpu.VMEM((_NCLS, 128), jnp.float32)],
        interpret=False,
    )(CLS, TXY, G)

    boxes = jnp.stack([ob[0, :_K], ob[1, :_K], ob[2, :_K], ob[3, :_K]],
                      axis=1)
    return boxes, os_[0, :_K], oc[0, :_K]
